# baseline (speedup 1.0000x reference)
import jax
import jax.numpy as jnp
from jax import lax
from jax.experimental import pallas as pl
from jax.experimental.pallas import tpu as pltpu
from jax.experimental.pallas import tpu_sc as plsc

_N = 16384
_D = 4
_L = 16
_NC, _NS = 2, 16
_NW = _NC * _NS
_FLAT = _N * _D
_CHUNK = _FLAT // _NW


def _body(w1_hbm, w2_hbm, out_hbm, w1_v, w2_v, out_v):
    wid = lax.axis_index("s") * _NC + lax.axis_index("c")
    pltpu.sync_copy(w1_hbm, w1_v)
    pltpu.sync_copy(w2_hbm, w2_v)
    p = jnp.sign(w1_v[...]) * jnp.sign(w2_v[...])
    for i in range(_CHUNK // _L):
        out_v[pl.ds(i * _L, _L)] = p
    pltpu.sync_copy(out_v, out_hbm.at[pl.ds(wid * _CHUNK, _CHUNK)])


@jax.jit
def _run(w1_tiled, w2_tiled):
    mesh = plsc.VectorSubcoreMesh(core_axis_name="c", subcore_axis_name="s")
    return pl.kernel(
        _body,
        out_type=jax.ShapeDtypeStruct((_FLAT,), jnp.float32),
        mesh=mesh,
        scratch_types=[
            pltpu.VMEM((_L,), jnp.float32),
            pltpu.VMEM((_L,), jnp.float32),
            pltpu.VMEM((_CHUNK,), jnp.float32),
        ],
    )(w1_tiled, w2_tiled)


def kernel(input, data1_weight, data2_weight):
    del input
    w1 = jnp.tile(data1_weight.reshape(_D), _L // _D)
    w2 = jnp.tile(data2_weight.reshape(_D), _L // _D)
    return _run(w1, w2).reshape(_N, _D)
